# baseline (speedup 1.0000x reference)
import jax
import jax.numpy as jnp
from jax import lax
from jax.experimental import pallas as pl
from jax.experimental.pallas import tpu as pltpu
from jax.experimental.pallas import tpu_sc as plsc

C = 128
NOUT = 25000
HALF = 12544
OUT_PAD = 2 * HALF
NTRASH = 64
ACC_ROWS = HALF + NTRASH
PER_TILE = HALF // 16
NC, NS, L = 2, 16, 16
NW = NC * NS
CH = 128
BLK = 512

KD, ED, ED_PAD = 8, 25000, 25088
BD = KD * ED_PAD
KS, ES, ES_PAD = 27, 15000, 15360
BS = 417792


def _sc_gather(table, idx4d, b_pad, br, nbuf, cols=C, dtype=jnp.float32):
    brch = br * CH
    nchb = b_pad // (NW * brch)
    nfull, tail = divmod(nchb, nbuf)

    def body(tbl, idx_hbm, out, idx_v, rows_v, *sems):
        gsems, wsems = sems[:nbuf], sems[nbuf:]
        c = lax.axis_index("c")
        s = lax.axis_index("s")
        wid = s * NC + c
        rbase = wid * (nchb * brch)
        pltpu.sync_copy(idx_hbm.at[wid], idx_v)
        for b in range(min(nbuf, nchb)):
            pltpu.async_copy(tbl.at[idx_v.at[b]], rows_v.at[b], gsems[b])

        def chunk(j, bb):
            pltpu.make_async_copy(tbl.at[idx_v.at[j]], rows_v.at[bb],
                                  gsems[bb]).wait()
            dst = out.at[pl.ds(rbase + j * brch, brch)]
            pltpu.async_copy(rows_v.at[bb], dst, wsems[bb])
            pltpu.make_async_copy(rows_v.at[bb], dst, wsems[bb]).wait()
            jn = j + nbuf

            @pl.when(jn < nchb)
            def _():
                pltpu.async_copy(tbl.at[idx_v.at[jn]], rows_v.at[bb],
                                 gsems[bb])

        def step(jb, carry):
            for b in range(nbuf):
                chunk(jb * nbuf + b, b)
            return carry

        lax.fori_loop(0, nfull, step, 0)
        for r in range(tail):
            chunk(nfull * nbuf + r, r)

    return pl.kernel(
        body,
        out_type=jax.ShapeDtypeStruct((b_pad, cols), dtype),
        mesh=plsc.VectorSubcoreMesh(core_axis_name="c", subcore_axis_name="s"),
        compiler_params=pltpu.CompilerParams(use_tc_tiling_on_sc=False),
        scratch_types=[
            pltpu.VMEM((nchb, brch), jnp.int32),
            pltpu.VMEM((nbuf, brch, cols), dtype),
        ] + [pltpu.SemaphoreType.DMA] * (2 * nbuf),
    )(table, idx4d)


SCCH = 64
IB = 8


def _sc_scatter(msgs, idx3d, zeros_tile, b_pad):
    nch = b_pad // (NS * SCCH)
    nbl_full, bl_tail = divmod(nch, IB)

    def body(msg_hbm, idx_hbm, zro, out, idx_v, li_v, msg_v, acc, *sems):
        lsems, msems, asems = sems[:2], sems[2:4], sems[4:6]
        c = lax.axis_index("c")
        s = lax.axis_index("s")
        row_base = c * HALF
        pltpu.sync_copy(zro, acc.at[pl.ds(s * PER_TILE, PER_TILE)])
        plsc.subcore_barrier()

        def idx_src(jb):
            return idx_hbm.at[s].at[pl.ds(jb * IB, IB)]

        def msg_src(j):
            return msg_hbm.at[pl.ds((s * nch + j) * SCCH, SCCH)]

        pltpu.async_copy(idx_src(0), idx_v.at[0], lsems[0])
        pltpu.async_copy(msg_src(0), msg_v.at[0], msems[0])
        pltpu.async_copy(msg_src(1), msg_v.at[1], msems[1])

        def block(jb, bl, nrows):
            pltpu.make_async_copy(idx_src(jb), idx_v.at[bl], lsems[bl]).wait()
            nb = jb + 1

            @pl.when(nb * IB < nch)
            def _():
                pltpu.async_copy(idx_src(nb), idx_v.at[1 - bl], lsems[1 - bl])

            for k in range(nrows):
                for t in range(SCCH // L):
                    v = idx_v[bl, k, pl.ds(t * L, L)]
                    li = v - row_base
                    ok = (li >= 0) & (li < HALF)
                    trash = HALF + (v & (NTRASH - 1))
                    li_v[bl, k, pl.ds(t * L, L)] = jnp.where(ok, li, trash)
            for k in range(nrows):
                j = jb * IB + k
                bb = k % 2
                pltpu.make_async_copy(msg_src(j), msg_v.at[bb],
                                      msems[bb]).wait()
                add = pltpu.async_copy(msg_v.at[bb],
                                       acc.at[li_v.at[bl].at[k]],
                                       asems[bb], add=True)
                add.wait()
                jn = j + 2

                @pl.when(jn < nch)
                def _():
                    pltpu.async_copy(msg_src(jn), msg_v.at[bb], msems[bb])

        def step2(jp, carry):
            block(jp * 2, 0, IB)
            block(jp * 2 + 1, 1, IB)
            return carry

        npair = nbl_full // 2
        lax.fori_loop(0, npair, step2, 0)
        if nbl_full % 2:
            block(nbl_full - 1, (nbl_full - 1) % 2, IB)
        if bl_tail:
            block(nbl_full, nbl_full % 2, bl_tail)
        plsc.subcore_barrier()
        nwb, wb_tail = divmod(PER_TILE, SCCH)
        for q in range(nwb):
            r = s * PER_TILE + q * SCCH
            pltpu.sync_copy(acc.at[pl.ds(r, SCCH)], msg_v.at[0])
            pltpu.sync_copy(msg_v.at[0], out.at[pl.ds(row_base + r, SCCH)])
        if wb_tail:
            r = s * PER_TILE + nwb * SCCH
            pltpu.sync_copy(acc.at[pl.ds(r, wb_tail)],
                            msg_v.at[0].at[pl.ds(0, wb_tail)])
            pltpu.sync_copy(msg_v.at[0].at[pl.ds(0, wb_tail)],
                            out.at[pl.ds(row_base + r, wb_tail)])

    return pl.kernel(
        body,
        out_type=jax.ShapeDtypeStruct((OUT_PAD, C), jnp.float32),
        mesh=plsc.VectorSubcoreMesh(core_axis_name="c", subcore_axis_name="s"),
        scratch_types=[
            pltpu.VMEM((2, IB, SCCH), jnp.int32),
            pltpu.VMEM((2, IB, SCCH), jnp.int32),
            pltpu.VMEM((2, SCCH, C), jnp.float32),
            pltpu.VMEM_SHARED((ACC_ROWS, C), jnp.float32),
        ] + [pltpu.SemaphoreType.DMA] * 6,
    )(msgs, idx3d, zeros_tile)


def _to_bits(t):
    b = t.astype(jnp.bfloat16).reshape(t.shape[0], C // 2, 2)
    return lax.bitcast_convert_type(b, jnp.int32)


def _from_bits(g, b_pad):
    return lax.bitcast_convert_type(g, jnp.bfloat16).reshape(b_pad, C)


def _tc_stats(raw, gb):
    nblk = OUT_PAD // BLK

    def body(raw_ref, gb_ref, out_ref, acc_ref):
        b = pl.program_id(0)

        @pl.when(b == 0)
        def _():
            acc_ref[...] = jnp.zeros_like(acc_ref)

        x = raw_ref[...]
        rows = b * BLK + lax.broadcasted_iota(jnp.int32, (BLK, C), 0)
        xm = jnp.where(rows < NOUT, x, 0.0)
        acc_ref[0:1, :] += jnp.sum(xm, axis=0, keepdims=True)
        acc_ref[1:2, :] += jnp.sum(xm * xm, axis=0, keepdims=True)

        @pl.when(b == nblk - 1)
        def _():
            mean = acc_ref[0:1, :] / NOUT
            var = acc_ref[1:2, :] / NOUT - mean * mean
            scale = gb_ref[0:1, :] * lax.rsqrt(var + 1e-5)
            shift = gb_ref[1:2, :] - mean * scale
            out_ref[...] = jnp.concatenate(
                [scale, shift, jnp.zeros((6, C), jnp.float32)], axis=0)

    return pl.pallas_call(
        body,
        grid=(nblk,),
        in_specs=[pl.BlockSpec((BLK, C), lambda b: (b, 0)),
                  pl.BlockSpec((8, C), lambda b: (0, 0))],
        out_specs=pl.BlockSpec((8, C), lambda b: (0, 0)),
        out_shape=jax.ShapeDtypeStruct((8, C), jnp.float32),
        scratch_shapes=[pltpu.VMEM((8, C), jnp.float32)],
    )(raw, gb)


def _tc_matmul(G, W, st, bpk, apply_act):
    B = G.shape[0]
    K = W.shape[0]
    nblk = B // BLK

    def body(g_ref, w_ref, st_ref, o_ref):
        g = g_ref[...].astype(jnp.float32)
        if apply_act:
            g = jnp.maximum(g * st_ref[0:1, :] + st_ref[1:2, :], 0.0)
        o_ref[...] = jnp.dot(g, w_ref[0], preferred_element_type=jnp.float32)

    return pl.pallas_call(
        body,
        grid=(nblk,),
        in_specs=[pl.BlockSpec((BLK, C), lambda b: (b, 0)),
                  pl.BlockSpec((1, C, C),
                               lambda b: (jnp.minimum(b // bpk, K - 1), 0, 0)),
                  pl.BlockSpec((8, C), lambda b: (0, 0))],
        out_specs=pl.BlockSpec((BLK, C), lambda b: (b, 0)),
        out_shape=jax.ShapeDtypeStruct((B, C), jnp.float32),
    )(G, W, st)


def _tc_final(raw2, P, st2, stp):
    nblk = OUT_PAD // BLK

    def body(a_ref, p_ref, s2_ref, sp_ref, o_ref):
        a = a_ref[...] * s2_ref[0:1, :] + s2_ref[1:2, :]
        q = p_ref[...] * sp_ref[0:1, :] + sp_ref[1:2, :]
        o_ref[...] = jnp.maximum(a + q, 0.0)

    return pl.pallas_call(
        body,
        grid=(nblk,),
        in_specs=[pl.BlockSpec((BLK, C), lambda b: (b, 0)),
                  pl.BlockSpec((BLK, C), lambda b: (b, 0)),
                  pl.BlockSpec((8, C), lambda b: (0, 0)),
                  pl.BlockSpec((8, C), lambda b: (0, 0))],
        out_specs=pl.BlockSpec((BLK, C), lambda b: (b, 0)),
        out_shape=jax.ShapeDtypeStruct((OUT_PAD, C), jnp.float32),
    )(raw2, P, st2, stp)


def _pad_idx(idx, e_pad, b_pad, fill, nw, width=CH, row_align=1):
    k, e = idx.shape
    p = jnp.pad(idx.astype(jnp.int32), ((0, 0), (0, e_pad - e)),
                constant_values=fill)
    flat = p.reshape(-1)
    flat = jnp.pad(flat, (0, b_pad - flat.shape[0]), constant_values=fill)
    a = flat.reshape(nw, b_pad // (nw * width), width)
    nch = a.shape[1]
    nch_pad = -(-nch // row_align) * row_align
    if nch_pad != nch:
        a = jnp.pad(a, ((0, 0), (0, nch_pad - nch), (0, 0)),
                    constant_values=fill)
    return a


def _pad_idx_g(idx, e_pad, b_pad, fill, br):
    a = _pad_idx(idx, e_pad, b_pad, fill, NW)
    return a.reshape(NW, a.shape[1] // br, br * CH)


def _gb(g, b):
    return jnp.concatenate([g[None], b[None], jnp.zeros((6, C), jnp.float32)], 0)


def kernel(x, down_in_idx, down_out_idx, sub_in_idx, sub_out_idx,
           W_down, W1, W2, W_proj,
           g_down, b_down, g1, b1, g2, b2, g_proj, b_proj):
    din = _pad_idx_g(down_in_idx, ED_PAD, BD, 0, br=1)
    dout = _pad_idx(down_out_idx, ED_PAD, BD, NOUT, NS, width=SCCH,
                    row_align=IB)
    sin = _pad_idx_g(sub_in_idx, ES_PAD, BS, 0, br=1)
    sout = _pad_idx(sub_out_idx, ES_PAD, BS, NOUT, NS, width=SCCH,
                    row_align=IB)
    zeros_tile = jnp.zeros((PER_TILE, C), jnp.float32)
    st0 = jnp.zeros((8, C), jnp.float32)

    Gd = _from_bits(_sc_gather(_to_bits(x), din, BD, br=1, nbuf=4,
                               cols=C // 2, dtype=jnp.int32), BD)
    Md = _tc_matmul(Gd, W_down, st0, ED_PAD // BLK, apply_act=False)
    raw_h = _sc_scatter(Md, dout, zeros_tile, BD)
    st_h = _tc_stats(raw_h, _gb(g_down, b_down))

    G1 = _from_bits(_sc_gather(_to_bits(raw_h), sin, BS, br=1, nbuf=6,
                               cols=C // 2, dtype=jnp.int32), BS)
    M1 = _tc_matmul(G1, W1, st_h, ES_PAD // BLK, apply_act=True)
    raw1 = _sc_scatter(M1, sout, zeros_tile, BS)
    st_1 = _tc_stats(raw1, _gb(g1, b1))

    G2 = _from_bits(_sc_gather(_to_bits(raw1), sin, BS, br=1, nbuf=6,
                               cols=C // 2, dtype=jnp.int32), BS)
    M2 = _tc_matmul(G2, W2, st_1, ES_PAD // BLK, apply_act=True)
    raw2 = _sc_scatter(M2, sout, zeros_tile, BS)
    st_2 = _tc_stats(raw2, _gb(g2, b2))

    P = _tc_matmul(raw_h, W_proj[None], st_h, OUT_PAD // BLK, apply_act=True)
    st_p = _tc_stats(P, _gb(g_proj, b_proj))

    outp = _tc_final(raw2, P, st_2, st_p)
    return outp[:NOUT]


# speedup vs baseline: 1.3473x; 1.3473x over previous
import jax
import jax.numpy as jnp
from jax import lax
from jax.experimental import pallas as pl
from jax.experimental.pallas import tpu as pltpu
from jax.experimental.pallas import tpu_sc as plsc

C = 128
NOUT = 25000
HALF = 12544
OUT_PAD = 2 * HALF
NTRASH = 64
ACC_ROWS = HALF + NTRASH
PER_TILE = HALF // 16
NC, NS, L = 2, 16, 16
NW = NC * NS
CH = 128
BLK = 512

KD, ED, ED_PAD = 8, 25000, 25088
BD = KD * ED_PAD
KS, ES, ES_PAD = 27, 15000, 15360
BS = 417792


def _sc_gather(table, idx4d, b_pad, br, nbuf, cols=C, dtype=jnp.float32):
    brch = br * CH
    nchb = b_pad // (NW * brch)
    nfull, tail = divmod(nchb, nbuf)

    def body(tbl, idx_hbm, out, idx_v, rows_v, *sems):
        gsems, wsems = sems[:nbuf], sems[nbuf:]
        c = lax.axis_index("c")
        s = lax.axis_index("s")
        wid = s * NC + c
        rbase = wid * (nchb * brch)
        pltpu.sync_copy(idx_hbm.at[wid], idx_v)
        for b in range(min(nbuf, nchb)):
            pltpu.async_copy(tbl.at[idx_v.at[b]], rows_v.at[b], gsems[b])

        def chunk(j, bb):
            pltpu.make_async_copy(tbl.at[idx_v.at[j]], rows_v.at[bb],
                                  gsems[bb]).wait()
            dst = out.at[pl.ds(rbase + j * brch, brch)]
            pltpu.async_copy(rows_v.at[bb], dst, wsems[bb])
            pltpu.make_async_copy(rows_v.at[bb], dst, wsems[bb]).wait()
            jn = j + nbuf

            @pl.when(jn < nchb)
            def _():
                pltpu.async_copy(tbl.at[idx_v.at[jn]], rows_v.at[bb],
                                 gsems[bb])

        def step(jb, carry):
            for b in range(nbuf):
                chunk(jb * nbuf + b, b)
            return carry

        lax.fori_loop(0, nfull, step, 0)
        for r in range(tail):
            chunk(nfull * nbuf + r, r)

    return pl.kernel(
        body,
        out_type=jax.ShapeDtypeStruct((b_pad, cols), dtype),
        mesh=plsc.VectorSubcoreMesh(core_axis_name="c", subcore_axis_name="s"),
        scratch_types=[
            pltpu.VMEM((nchb, brch), jnp.int32),
            pltpu.VMEM((nbuf, brch, cols), dtype),
        ] + [pltpu.SemaphoreType.DMA] * (2 * nbuf),
    )(table, idx4d)


TBL_ROWS = OUT_PAD // NS
CB = C // 2


def _sc_gather_sp(table_bits, idx4d, b_pad):
    nchb = b_pad // (NW * CH)
    nbuf = 2
    nfull, tail = divmod(nchb, nbuf)

    def body(tbl, idx_hbm, out, idx_v, rows_v, tbl_sh, *sems):
        gsems, wsems = sems[:nbuf], sems[nbuf:]
        c = lax.axis_index("c")
        s = lax.axis_index("s")
        wid = s * NC + c
        rbase = wid * (nchb * CH)
        for q in range(TBL_ROWS // 112):
            r = s * TBL_ROWS + q * 112
            stage = rows_v.at[0].at[pl.ds(0, 112)]
            pltpu.sync_copy(tbl.at[pl.ds(r, 112)], stage)
            pltpu.sync_copy(stage, tbl_sh.at[pl.ds(r, 112)])
        pltpu.sync_copy(idx_hbm.at[wid], idx_v)
        plsc.subcore_barrier()
        for b in range(nbuf):
            pltpu.async_copy(tbl_sh.at[idx_v.at[b]], rows_v.at[b], gsems[b])

        def chunk(j, bb):
            pltpu.make_async_copy(tbl_sh.at[idx_v.at[j]], rows_v.at[bb],
                                  gsems[bb]).wait()
            dst = out.at[pl.ds(rbase + j * CH, CH)]
            pltpu.async_copy(rows_v.at[bb], dst, wsems[bb])
            pltpu.make_async_copy(rows_v.at[bb], dst, wsems[bb]).wait()
            jn = j + nbuf

            @pl.when(jn < nchb)
            def _():
                pltpu.async_copy(tbl_sh.at[idx_v.at[jn]], rows_v.at[bb],
                                 gsems[bb])

        def step(jb, carry):
            for b in range(nbuf):
                chunk(jb * nbuf + b, b)
            return carry

        lax.fori_loop(0, nfull, step, 0)
        for r in range(tail):
            chunk(nfull * nbuf + r, r)

    return pl.kernel(
        body,
        out_type=jax.ShapeDtypeStruct((b_pad, CB), jnp.int32),
        mesh=plsc.VectorSubcoreMesh(core_axis_name="c", subcore_axis_name="s"),
        compiler_params=pltpu.CompilerParams(use_tc_tiling_on_sc=False),
        scratch_types=[
            pltpu.VMEM((nchb, CH), jnp.int32),
            pltpu.VMEM((nbuf, CH, CB), jnp.int32),
            pltpu.VMEM_SHARED((OUT_PAD, CB), jnp.int32),
        ] + [pltpu.SemaphoreType.DMA] * (2 * nbuf),
    )(table_bits, idx4d)


SCCH = 64
IB = 8


def _sc_scatter(msgs, idx3d, zeros_tile, b_pad):
    nch = b_pad // (NS * SCCH)
    nbl_full, bl_tail = divmod(nch, IB)

    def body(msg_hbm, idx_hbm, zro, out, idx_v, li_v, msg_v, acc, *sems):
        lsems, msems, asems = sems[:2], sems[2:4], sems[4:6]
        c = lax.axis_index("c")
        s = lax.axis_index("s")
        row_base = c * HALF
        pltpu.sync_copy(zro, acc.at[pl.ds(s * PER_TILE, PER_TILE)])
        plsc.subcore_barrier()

        def idx_src(jb):
            return idx_hbm.at[s].at[pl.ds(jb * IB, IB)]

        def msg_src(j):
            return msg_hbm.at[pl.ds((s * nch + j) * SCCH, SCCH)]

        pltpu.async_copy(idx_src(0), idx_v.at[0], lsems[0])
        pltpu.async_copy(msg_src(0), msg_v.at[0], msems[0])
        pltpu.async_copy(msg_src(1), msg_v.at[1], msems[1])

        def block(jb, bl, nrows):
            pltpu.make_async_copy(idx_src(jb), idx_v.at[bl], lsems[bl]).wait()
            nb = jb + 1

            @pl.when(nb * IB < nch)
            def _():
                pltpu.async_copy(idx_src(nb), idx_v.at[1 - bl], lsems[1 - bl])

            for k in range(nrows):
                for t in range(SCCH // L):
                    v = idx_v[bl, k, pl.ds(t * L, L)]
                    li = v - row_base
                    ok = (li >= 0) & (li < HALF)
                    trash = HALF + (v & (NTRASH - 1))
                    li_v[bl, k, pl.ds(t * L, L)] = jnp.where(ok, li, trash)
            for k in range(nrows):
                j = jb * IB + k
                bb = k % 2
                pltpu.make_async_copy(msg_src(j), msg_v.at[bb],
                                      msems[bb]).wait()
                add = pltpu.async_copy(msg_v.at[bb],
                                       acc.at[li_v.at[bl].at[k]],
                                       asems[bb], add=True)
                add.wait()
                jn = j + 2

                @pl.when(jn < nch)
                def _():
                    pltpu.async_copy(msg_src(jn), msg_v.at[bb], msems[bb])

        def step2(jp, carry):
            block(jp * 2, 0, IB)
            block(jp * 2 + 1, 1, IB)
            return carry

        npair = nbl_full // 2
        lax.fori_loop(0, npair, step2, 0)
        if nbl_full % 2:
            block(nbl_full - 1, (nbl_full - 1) % 2, IB)
        if bl_tail:
            block(nbl_full, nbl_full % 2, bl_tail)
        plsc.subcore_barrier()
        nwb, wb_tail = divmod(PER_TILE, SCCH)
        for q in range(nwb):
            r = s * PER_TILE + q * SCCH
            pltpu.sync_copy(acc.at[pl.ds(r, SCCH)], msg_v.at[0])
            pltpu.sync_copy(msg_v.at[0], out.at[pl.ds(row_base + r, SCCH)])
        if wb_tail:
            r = s * PER_TILE + nwb * SCCH
            pltpu.sync_copy(acc.at[pl.ds(r, wb_tail)],
                            msg_v.at[0].at[pl.ds(0, wb_tail)])
            pltpu.sync_copy(msg_v.at[0].at[pl.ds(0, wb_tail)],
                            out.at[pl.ds(row_base + r, wb_tail)])

    return pl.kernel(
        body,
        out_type=jax.ShapeDtypeStruct((OUT_PAD, C), jnp.float32),
        mesh=plsc.VectorSubcoreMesh(core_axis_name="c", subcore_axis_name="s"),
        scratch_types=[
            pltpu.VMEM((2, IB, SCCH), jnp.int32),
            pltpu.VMEM((2, IB, SCCH), jnp.int32),
            pltpu.VMEM((2, SCCH, C), jnp.float32),
            pltpu.VMEM_SHARED((ACC_ROWS, C), jnp.float32),
        ] + [pltpu.SemaphoreType.DMA] * 6,
    )(msgs, idx3d, zeros_tile)


def _to_bits(t):
    b = t.astype(jnp.bfloat16).reshape(t.shape[0], C // 2, 2)
    return lax.bitcast_convert_type(b, jnp.int32)


def _from_bits(g, b_pad):
    return lax.bitcast_convert_type(g, jnp.bfloat16).reshape(b_pad, C)


def _tc_stats(raw, gb):
    nblk = OUT_PAD // BLK

    def body(raw_ref, gb_ref, out_ref, acc_ref):
        b = pl.program_id(0)

        @pl.when(b == 0)
        def _():
            acc_ref[...] = jnp.zeros_like(acc_ref)

        x = raw_ref[...]
        rows = b * BLK + lax.broadcasted_iota(jnp.int32, (BLK, C), 0)
        xm = jnp.where(rows < NOUT, x, 0.0)
        acc_ref[0:1, :] += jnp.sum(xm, axis=0, keepdims=True)
        acc_ref[1:2, :] += jnp.sum(xm * xm, axis=0, keepdims=True)

        @pl.when(b == nblk - 1)
        def _():
            mean = acc_ref[0:1, :] / NOUT
            var = acc_ref[1:2, :] / NOUT - mean * mean
            scale = gb_ref[0:1, :] * lax.rsqrt(var + 1e-5)
            shift = gb_ref[1:2, :] - mean * scale
            out_ref[...] = jnp.concatenate(
                [scale, shift, jnp.zeros((6, C), jnp.float32)], axis=0)

    return pl.pallas_call(
        body,
        grid=(nblk,),
        in_specs=[pl.BlockSpec((BLK, C), lambda b: (b, 0)),
                  pl.BlockSpec((8, C), lambda b: (0, 0))],
        out_specs=pl.BlockSpec((8, C), lambda b: (0, 0)),
        out_shape=jax.ShapeDtypeStruct((8, C), jnp.float32),
        scratch_shapes=[pltpu.VMEM((8, C), jnp.float32)],
    )(raw, gb)


def _tc_matmul(G, W, st, bpk, apply_act):
    B = G.shape[0]
    K = W.shape[0]
    nblk = B // BLK

    def body(g_ref, w_ref, st_ref, o_ref):
        g = g_ref[...].astype(jnp.float32)
        if apply_act:
            g = jnp.maximum(g * st_ref[0:1, :] + st_ref[1:2, :], 0.0)
        o_ref[...] = jnp.dot(g, w_ref[0], preferred_element_type=jnp.float32)

    return pl.pallas_call(
        body,
        grid=(nblk,),
        in_specs=[pl.BlockSpec((BLK, C), lambda b: (b, 0)),
                  pl.BlockSpec((1, C, C),
                               lambda b: (jnp.minimum(b // bpk, K - 1), 0, 0)),
                  pl.BlockSpec((8, C), lambda b: (0, 0))],
        out_specs=pl.BlockSpec((BLK, C), lambda b: (b, 0)),
        out_shape=jax.ShapeDtypeStruct((B, C), jnp.float32),
    )(G, W, st)


def _tc_final(raw2, P, st2, stp):
    nblk = OUT_PAD // BLK

    def body(a_ref, p_ref, s2_ref, sp_ref, o_ref):
        a = a_ref[...] * s2_ref[0:1, :] + s2_ref[1:2, :]
        q = p_ref[...] * sp_ref[0:1, :] + sp_ref[1:2, :]
        o_ref[...] = jnp.maximum(a + q, 0.0)

    return pl.pallas_call(
        body,
        grid=(nblk,),
        in_specs=[pl.BlockSpec((BLK, C), lambda b: (b, 0)),
                  pl.BlockSpec((BLK, C), lambda b: (b, 0)),
                  pl.BlockSpec((8, C), lambda b: (0, 0)),
                  pl.BlockSpec((8, C), lambda b: (0, 0))],
        out_specs=pl.BlockSpec((BLK, C), lambda b: (b, 0)),
        out_shape=jax.ShapeDtypeStruct((OUT_PAD, C), jnp.float32),
    )(raw2, P, st2, stp)


def _pad_idx(idx, e_pad, b_pad, fill, nw, width=CH, row_align=1):
    k, e = idx.shape
    p = jnp.pad(idx.astype(jnp.int32), ((0, 0), (0, e_pad - e)),
                constant_values=fill)
    flat = p.reshape(-1)
    flat = jnp.pad(flat, (0, b_pad - flat.shape[0]), constant_values=fill)
    a = flat.reshape(nw, b_pad // (nw * width), width)
    nch = a.shape[1]
    nch_pad = -(-nch // row_align) * row_align
    if nch_pad != nch:
        a = jnp.pad(a, ((0, 0), (0, nch_pad - nch), (0, 0)),
                    constant_values=fill)
    return a


def _pad_idx_g(idx, e_pad, b_pad, fill, br):
    a = _pad_idx(idx, e_pad, b_pad, fill, NW)
    return a.reshape(NW, a.shape[1] // br, br * CH)


def _gb(g, b):
    return jnp.concatenate([g[None], b[None], jnp.zeros((6, C), jnp.float32)], 0)


def kernel(x, down_in_idx, down_out_idx, sub_in_idx, sub_out_idx,
           W_down, W1, W2, W_proj,
           g_down, b_down, g1, b1, g2, b2, g_proj, b_proj):
    din = _pad_idx_g(down_in_idx, ED_PAD, BD, 0, br=1)
    dout = _pad_idx(down_out_idx, ED_PAD, BD, NOUT, NS, width=SCCH,
                    row_align=IB)
    sin = _pad_idx_g(sub_in_idx, ES_PAD, BS, 0, br=1)
    sout = _pad_idx(sub_out_idx, ES_PAD, BS, NOUT, NS, width=SCCH,
                    row_align=IB)
    zeros_tile = jnp.zeros((PER_TILE, C), jnp.float32)
    st0 = jnp.zeros((8, C), jnp.float32)

    Gd = _sc_gather(x, din, BD, br=1, nbuf=4)
    Md = _tc_matmul(Gd, W_down, st0, ED_PAD // BLK, apply_act=False)
    raw_h = _sc_scatter(Md, dout, zeros_tile, BD)
    st_h = _tc_stats(raw_h, _gb(g_down, b_down))

    G1 = _from_bits(_sc_gather_sp(_to_bits(raw_h), sin, BS), BS)
    M1 = _tc_matmul(G1, W1, st_h, ES_PAD // BLK, apply_act=True)
    raw1 = _sc_scatter(M1, sout, zeros_tile, BS)
    st_1 = _tc_stats(raw1, _gb(g1, b1))

    G2 = _from_bits(_sc_gather_sp(_to_bits(raw1), sin, BS), BS)
    M2 = _tc_matmul(G2, W2, st_1, ES_PAD // BLK, apply_act=True)
    raw2 = _sc_scatter(M2, sout, zeros_tile, BS)
    st_2 = _tc_stats(raw2, _gb(g2, b2))

    P = _tc_matmul(raw_h, W_proj[None], st_h, OUT_PAD // BLK, apply_act=True)
    st_p = _tc_stats(P, _gb(g_proj, b_proj))

    outp = _tc_final(raw2, P, st_2, st_p)
    return outp[:NOUT]


# speedup vs baseline: 2.1233x; 1.5760x over previous
import jax
import jax.numpy as jnp
from jax import lax
from jax.experimental import pallas as pl
from jax.experimental.pallas import tpu as pltpu
from jax.experimental.pallas import tpu_sc as plsc

C = 128
NOUT = 25000
HALF = 12544
OUT_PAD = 2 * HALF
NTRASH = 64
ACC_ROWS = HALF + NTRASH
PER_TILE = HALF // 16
NC, NS, L = 2, 16, 16
NW = NC * NS
CH = 128
BLK = 512

KD, ED, ED_PAD = 8, 25000, 25088
BD = KD * ED_PAD
KS, ES, ES_PAD = 27, 15000, 15360
BS = 417792


def _sc_gather(table, idx4d, b_pad, br, nbuf, cols=C, dtype=jnp.float32):
    brch = br * CH
    nchb = b_pad // (NW * brch)
    nfull, tail = divmod(nchb, nbuf)

    def body(tbl, idx_hbm, out, idx_v, rows_v, *sems):
        gsems, wsems = sems[:nbuf], sems[nbuf:]
        c = lax.axis_index("c")
        s = lax.axis_index("s")
        wid = s * NC + c
        rbase = wid * (nchb * brch)
        pltpu.sync_copy(idx_hbm.at[wid], idx_v)
        for b in range(min(nbuf, nchb)):
            pltpu.async_copy(tbl.at[idx_v.at[b]], rows_v.at[b], gsems[b])

        def chunk(j, bb):
            pltpu.make_async_copy(tbl.at[idx_v.at[j]], rows_v.at[bb],
                                  gsems[bb]).wait()
            dst = out.at[pl.ds(rbase + j * brch, brch)]
            pltpu.async_copy(rows_v.at[bb], dst, wsems[bb])
            pltpu.make_async_copy(rows_v.at[bb], dst, wsems[bb]).wait()
            jn = j + nbuf

            @pl.when(jn < nchb)
            def _():
                pltpu.async_copy(tbl.at[idx_v.at[jn]], rows_v.at[bb],
                                 gsems[bb])

        def step(jb, carry):
            for b in range(nbuf):
                chunk(jb * nbuf + b, b)
            return carry

        lax.fori_loop(0, nfull, step, 0)
        for r in range(tail):
            chunk(nfull * nbuf + r, r)

    return pl.kernel(
        body,
        out_type=jax.ShapeDtypeStruct((b_pad, cols), dtype),
        mesh=plsc.VectorSubcoreMesh(core_axis_name="c", subcore_axis_name="s"),
        scratch_types=[
            pltpu.VMEM((nchb, brch), jnp.int32),
            pltpu.VMEM((nbuf, brch, cols), dtype),
        ] + [pltpu.SemaphoreType.DMA] * (2 * nbuf),
    )(table, idx4d)


TBL_ROWS = OUT_PAD // NS
CB = C // 2


def _sc_gather_sp(table_bits, idx4d, b_pad):
    nchb = b_pad // (NW * CH)
    nbuf = 2
    nfull, tail = divmod(nchb, nbuf)

    def body(tbl, idx_hbm, out, idx_v, rows_v, tbl_sh, *sems):
        gsems, wsems = sems[:nbuf], sems[nbuf:]
        c = lax.axis_index("c")
        s = lax.axis_index("s")
        wid = s * NC + c
        rbase = wid * (nchb * CH)
        for q in range(TBL_ROWS // 112):
            r = s * TBL_ROWS + q * 112
            stage = rows_v.at[0].at[pl.ds(0, 112)]
            pltpu.sync_copy(tbl.at[pl.ds(r, 112)], stage)
            pltpu.sync_copy(stage, tbl_sh.at[pl.ds(r, 112)])
        pltpu.sync_copy(idx_hbm.at[wid], idx_v)
        plsc.subcore_barrier()
        for b in range(nbuf):
            pltpu.async_copy(tbl_sh.at[idx_v.at[b]], rows_v.at[b], gsems[b])

        def chunk(j, bb):
            pltpu.make_async_copy(tbl_sh.at[idx_v.at[j]], rows_v.at[bb],
                                  gsems[bb]).wait()
            dst = out.at[pl.ds(rbase + j * CH, CH)]
            pltpu.async_copy(rows_v.at[bb], dst, wsems[bb])
            pltpu.make_async_copy(rows_v.at[bb], dst, wsems[bb]).wait()
            jn = j + nbuf

            @pl.when(jn < nchb)
            def _():
                pltpu.async_copy(tbl_sh.at[idx_v.at[jn]], rows_v.at[bb],
                                 gsems[bb])

        def step(jb, carry):
            for b in range(nbuf):
                chunk(jb * nbuf + b, b)
            return carry

        lax.fori_loop(0, nfull, step, 0)
        for r in range(tail):
            chunk(nfull * nbuf + r, r)

    return pl.kernel(
        body,
        out_type=jax.ShapeDtypeStruct((b_pad, CB), jnp.int32),
        mesh=plsc.VectorSubcoreMesh(core_axis_name="c", subcore_axis_name="s"),
        compiler_params=pltpu.CompilerParams(use_tc_tiling_on_sc=False),
        scratch_types=[
            pltpu.VMEM((nchb, CH), jnp.int32),
            pltpu.VMEM((nbuf, CH, CB), jnp.int32),
            pltpu.VMEM_SHARED((OUT_PAD, CB), jnp.int32),
        ] + [pltpu.SemaphoreType.DMA] * (2 * nbuf),
    )(table_bits, idx4d)


SCCH = 64
IB = 8


def _sc_scatter(msgs, idx3d, zeros_tile, b_pad):
    nch = b_pad // (NS * SCCH)
    nbl_full, bl_tail = divmod(nch, IB)

    def body(msg_hbm, idx_hbm, zro, out, idx_v, li_v, msg_v, acc, *sems):
        lsems, msems, asems = sems[:2], sems[2:4], sems[4:6]
        c = lax.axis_index("c")
        s = lax.axis_index("s")
        row_base = c * HALF
        pltpu.sync_copy(zro, acc.at[pl.ds(s * PER_TILE, PER_TILE)])
        plsc.subcore_barrier()

        def idx_src(jb):
            return idx_hbm.at[s].at[pl.ds(jb * IB, IB)]

        def msg_src(j):
            return msg_hbm.at[pl.ds((s * nch + j) * SCCH, SCCH)]

        pltpu.async_copy(idx_src(0), idx_v.at[0], lsems[0])
        pltpu.async_copy(msg_src(0), msg_v.at[0], msems[0])
        pltpu.async_copy(msg_src(1), msg_v.at[1], msems[1])

        def block(jb, bl, nrows):
            pltpu.make_async_copy(idx_src(jb), idx_v.at[bl], lsems[bl]).wait()
            nb = jb + 1

            @pl.when(nb * IB < nch)
            def _():
                pltpu.async_copy(idx_src(nb), idx_v.at[1 - bl], lsems[1 - bl])

            for k in range(nrows):
                for t in range(SCCH // L):
                    v = idx_v[bl, k, pl.ds(t * L, L)]
                    li = v - row_base
                    ok = (li >= 0) & (li < HALF)
                    trash = HALF + (v & (NTRASH - 1))
                    li_v[bl, k, pl.ds(t * L, L)] = jnp.where(ok, li, trash)
            for k in range(nrows):
                j = jb * IB + k
                bb = k % 2
                pltpu.make_async_copy(msg_src(j), msg_v.at[bb],
                                      msems[bb]).wait()
                add = pltpu.async_copy(msg_v.at[bb],
                                       acc.at[li_v.at[bl].at[k]],
                                       asems[bb], add=True)
                add.wait()
                jn = j + 2

                @pl.when(jn < nch)
                def _():
                    pltpu.async_copy(msg_src(jn), msg_v.at[bb], msems[bb])

        def step2(jp, carry):
            block(jp * 2, 0, IB)
            block(jp * 2 + 1, 1, IB)
            return carry

        npair = nbl_full // 2
        lax.fori_loop(0, npair, step2, 0)
        if nbl_full % 2:
            block(nbl_full - 1, (nbl_full - 1) % 2, IB)
        if bl_tail:
            block(nbl_full, nbl_full % 2, bl_tail)
        plsc.subcore_barrier()
        nwb, wb_tail = divmod(PER_TILE, SCCH)
        for q in range(nwb):
            r = s * PER_TILE + q * SCCH
            pltpu.sync_copy(acc.at[pl.ds(r, SCCH)], msg_v.at[0])
            pltpu.sync_copy(msg_v.at[0], out.at[pl.ds(row_base + r, SCCH)])
        if wb_tail:
            r = s * PER_TILE + nwb * SCCH
            pltpu.sync_copy(acc.at[pl.ds(r, wb_tail)],
                            msg_v.at[0].at[pl.ds(0, wb_tail)])
            pltpu.sync_copy(msg_v.at[0].at[pl.ds(0, wb_tail)],
                            out.at[pl.ds(row_base + r, wb_tail)])

    return pl.kernel(
        body,
        out_type=jax.ShapeDtypeStruct((OUT_PAD, C), jnp.float32),
        mesh=plsc.VectorSubcoreMesh(core_axis_name="c", subcore_axis_name="s"),
        scratch_types=[
            pltpu.VMEM((2, IB, SCCH), jnp.int32),
            pltpu.VMEM((2, IB, SCCH), jnp.int32),
            pltpu.VMEM((2, SCCH, C), jnp.float32),
            pltpu.VMEM_SHARED((ACC_ROWS, C), jnp.float32),
        ] + [pltpu.SemaphoreType.DMA] * 6,
    )(msgs, idx3d, zeros_tile)


def _to_bits(t):
    b = t.astype(jnp.bfloat16).reshape(t.shape[0], C // 2, 2)
    return lax.bitcast_convert_type(b, jnp.int32)


def _from_bits(g, b_pad):
    return lax.bitcast_convert_type(g, jnp.bfloat16).reshape(b_pad, C)


def _tc_stats(raw, gb):
    nblk = OUT_PAD // BLK

    def body(raw_ref, gb_ref, out_ref, acc_ref):
        b = pl.program_id(0)

        @pl.when(b == 0)
        def _():
            acc_ref[...] = jnp.zeros_like(acc_ref)

        x = raw_ref[...]
        rows = b * BLK + lax.broadcasted_iota(jnp.int32, (BLK, C), 0)
        xm = jnp.where(rows < NOUT, x, 0.0)
        acc_ref[0:1, :] += jnp.sum(xm, axis=0, keepdims=True)
        acc_ref[1:2, :] += jnp.sum(xm * xm, axis=0, keepdims=True)

        @pl.when(b == nblk - 1)
        def _():
            mean = acc_ref[0:1, :] / NOUT
            var = acc_ref[1:2, :] / NOUT - mean * mean
            scale = gb_ref[0:1, :] * lax.rsqrt(var + 1e-5)
            shift = gb_ref[1:2, :] - mean * scale
            out_ref[...] = jnp.concatenate(
                [scale, shift, jnp.zeros((6, C), jnp.float32)], axis=0)

    return pl.pallas_call(
        body,
        grid=(nblk,),
        in_specs=[pl.BlockSpec((BLK, C), lambda b: (b, 0)),
                  pl.BlockSpec((8, C), lambda b: (0, 0))],
        out_specs=pl.BlockSpec((8, C), lambda b: (0, 0)),
        out_shape=jax.ShapeDtypeStruct((8, C), jnp.float32),
        scratch_shapes=[pltpu.VMEM((8, C), jnp.float32)],
    )(raw, gb)


def _tc_matmul(G, W, st, bpk, apply_act):
    B = G.shape[0]
    K = W.shape[0]
    nblk = B // BLK

    def body(g_ref, w_ref, st_ref, o_ref):
        g = g_ref[...].astype(jnp.float32)
        if apply_act:
            g = jnp.maximum(g * st_ref[0:1, :] + st_ref[1:2, :], 0.0)
        o_ref[...] = jnp.dot(g, w_ref[0], preferred_element_type=jnp.float32)

    return pl.pallas_call(
        body,
        grid=(nblk,),
        in_specs=[pl.BlockSpec((BLK, C), lambda b: (b, 0)),
                  pl.BlockSpec((1, C, C),
                               lambda b: (jnp.minimum(b // bpk, K - 1), 0, 0)),
                  pl.BlockSpec((8, C), lambda b: (0, 0))],
        out_specs=pl.BlockSpec((BLK, C), lambda b: (b, 0)),
        out_shape=jax.ShapeDtypeStruct((B, C), jnp.float32),
    )(G, W, st)


def _tc_matmul_packed(gbits, wp, stp, bpk):
    B = gbits.shape[0]
    K = wp.shape[0]
    nblk = B // BLK

    def body(g_ref, w_ref, st_ref, o_ref):
        x = g_ref[...]
        lo = lax.bitcast_convert_type(lax.shift_left(x, 16), jnp.float32)
        hi = lax.bitcast_convert_type(x & jnp.int32(-65536), jnp.float32)
        ge = jnp.maximum(lo * st_ref[0:1, 0:64] + st_ref[1:2, 0:64], 0.0)
        go = jnp.maximum(hi * st_ref[2:3, 0:64] + st_ref[3:4, 0:64], 0.0)
        o_ref[...] = (
            jnp.dot(ge, w_ref[0, 0], preferred_element_type=jnp.float32)
            + jnp.dot(go, w_ref[0, 1], preferred_element_type=jnp.float32))

    return pl.pallas_call(
        body,
        grid=(nblk,),
        in_specs=[pl.BlockSpec((BLK, C // 2), lambda b: (b, 0)),
                  pl.BlockSpec((1, 2, C // 2, C),
                               lambda b: (jnp.minimum(b // bpk, K - 1), 0, 0, 0)),
                  pl.BlockSpec((8, C), lambda b: (0, 0))],
        out_specs=pl.BlockSpec((BLK, C), lambda b: (b, 0)),
        out_shape=jax.ShapeDtypeStruct((B, C), jnp.float32),
    )(gbits, wp, stp)


def _split_w(W):
    return jnp.stack([W[:, 0::2, :], W[:, 1::2, :]], axis=1)


def _split_st(st):
    top = jnp.stack([st[0, 0::2], st[1, 0::2], st[0, 1::2], st[1, 1::2]])
    return jnp.pad(top, ((0, 4), (0, C // 2)))


def _tc_final(raw2, P, st2, stp):
    nblk = OUT_PAD // BLK

    def body(a_ref, p_ref, s2_ref, sp_ref, o_ref):
        a = a_ref[...] * s2_ref[0:1, :] + s2_ref[1:2, :]
        q = p_ref[...] * sp_ref[0:1, :] + sp_ref[1:2, :]
        o_ref[...] = jnp.maximum(a + q, 0.0)

    return pl.pallas_call(
        body,
        grid=(nblk,),
        in_specs=[pl.BlockSpec((BLK, C), lambda b: (b, 0)),
                  pl.BlockSpec((BLK, C), lambda b: (b, 0)),
                  pl.BlockSpec((8, C), lambda b: (0, 0)),
                  pl.BlockSpec((8, C), lambda b: (0, 0))],
        out_specs=pl.BlockSpec((BLK, C), lambda b: (b, 0)),
        out_shape=jax.ShapeDtypeStruct((OUT_PAD, C), jnp.float32),
    )(raw2, P, st2, stp)


def _pad_idx(idx, e_pad, b_pad, fill, nw, width=CH, row_align=1):
    k, e = idx.shape
    p = jnp.pad(idx.astype(jnp.int32), ((0, 0), (0, e_pad - e)),
                constant_values=fill)
    flat = p.reshape(-1)
    flat = jnp.pad(flat, (0, b_pad - flat.shape[0]), constant_values=fill)
    a = flat.reshape(nw, b_pad // (nw * width), width)
    nch = a.shape[1]
    nch_pad = -(-nch // row_align) * row_align
    if nch_pad != nch:
        a = jnp.pad(a, ((0, 0), (0, nch_pad - nch), (0, 0)),
                    constant_values=fill)
    return a


def _pad_idx_g(idx, e_pad, b_pad, fill, br):
    a = _pad_idx(idx, e_pad, b_pad, fill, NW)
    return a.reshape(NW, a.shape[1] // br, br * CH)


def _gb(g, b):
    return jnp.concatenate([g[None], b[None], jnp.zeros((6, C), jnp.float32)], 0)


def kernel(x, down_in_idx, down_out_idx, sub_in_idx, sub_out_idx,
           W_down, W1, W2, W_proj,
           g_down, b_down, g1, b1, g2, b2, g_proj, b_proj):
    din = _pad_idx_g(down_in_idx, ED_PAD, BD, 0, br=1)
    dout = _pad_idx(down_out_idx, ED_PAD, BD, NOUT, NS, width=SCCH,
                    row_align=IB)
    sin = _pad_idx_g(sub_in_idx, ES_PAD, BS, 0, br=1)
    sout = _pad_idx(sub_out_idx, ES_PAD, BS, NOUT, NS, width=SCCH,
                    row_align=IB)
    zeros_tile = jnp.zeros((PER_TILE, C), jnp.float32)
    st0 = jnp.zeros((8, C), jnp.float32)

    Gd = _sc_gather(x, din, BD, br=1, nbuf=4)
    Md = _tc_matmul(Gd, W_down, st0, ED_PAD // BLK, apply_act=False)
    raw_h = _sc_scatter(Md, dout, zeros_tile, BD)
    st_h = _tc_stats(raw_h, _gb(g_down, b_down))

    G1 = _sc_gather_sp(_to_bits(raw_h), sin, BS)
    M1 = _tc_matmul_packed(G1, _split_w(W1), _split_st(st_h), ES_PAD // BLK)
    raw1 = _sc_scatter(M1, sout, zeros_tile, BS)
    st_1 = _tc_stats(raw1, _gb(g1, b1))

    G2 = _sc_gather_sp(_to_bits(raw1), sin, BS)
    M2 = _tc_matmul_packed(G2, _split_w(W2), _split_st(st_1), ES_PAD // BLK)
    raw2 = _sc_scatter(M2, sout, zeros_tile, BS)
    st_2 = _tc_stats(raw2, _gb(g2, b2))

    P = _tc_matmul(raw_h, W_proj[None], st_h, OUT_PAD // BLK, apply_act=True)
    st_p = _tc_stats(P, _gb(g_proj, b_proj))

    outp = _tc_final(raw2, P, st_2, st_p)
    return outp[:NOUT]


# speedup vs baseline: 2.2740x; 1.0709x over previous
import jax
import jax.numpy as jnp
from jax import lax
from jax.experimental import pallas as pl
from jax.experimental.pallas import tpu as pltpu
from jax.experimental.pallas import tpu_sc as plsc

C = 128
NOUT = 25000
HALF = 12544
OUT_PAD = 2 * HALF
NTRASH = 64
ACC_ROWS = HALF + NTRASH
PER_TILE = HALF // 16
NC, NS, L = 2, 16, 16
NW = NC * NS
CH = 128
BLK = 512

KD, ED, ED_PAD = 8, 25000, 25088
BD = KD * ED_PAD
KS, ES, ES_PAD = 27, 15000, 15360
BS = 417792


def _sc_gather(table, idx4d, b_pad, br, nbuf, cols=C, dtype=jnp.float32):
    brch = br * CH
    nchb = b_pad // (NW * brch)
    nfull, tail = divmod(nchb, nbuf)

    def body(tbl, idx_hbm, out, idx_v, rows_v, *sems):
        gsems, wsems = sems[:nbuf], sems[nbuf:]
        c = lax.axis_index("c")
        s = lax.axis_index("s")
        wid = s * NC + c
        rbase = wid * (nchb * brch)
        pltpu.sync_copy(idx_hbm.at[wid], idx_v)
        for b in range(min(nbuf, nchb)):
            pltpu.async_copy(tbl.at[idx_v.at[b]], rows_v.at[b], gsems[b])

        def chunk(j, bb):
            pltpu.make_async_copy(tbl.at[idx_v.at[j]], rows_v.at[bb],
                                  gsems[bb]).wait()
            dst = out.at[pl.ds(rbase + j * brch, brch)]
            pltpu.async_copy(rows_v.at[bb], dst, wsems[bb])
            pltpu.make_async_copy(rows_v.at[bb], dst, wsems[bb]).wait()
            jn = j + nbuf

            @pl.when(jn < nchb)
            def _():
                pltpu.async_copy(tbl.at[idx_v.at[jn]], rows_v.at[bb],
                                 gsems[bb])

        def step(jb, carry):
            for b in range(nbuf):
                chunk(jb * nbuf + b, b)
            return carry

        lax.fori_loop(0, nfull, step, 0)
        for r in range(tail):
            chunk(nfull * nbuf + r, r)

    return pl.kernel(
        body,
        out_type=jax.ShapeDtypeStruct((b_pad, cols), dtype),
        mesh=plsc.VectorSubcoreMesh(core_axis_name="c", subcore_axis_name="s"),
        scratch_types=[
            pltpu.VMEM((nchb, brch), jnp.int32),
            pltpu.VMEM((nbuf, brch, cols), dtype),
        ] + [pltpu.SemaphoreType.DMA] * (2 * nbuf),
    )(table, idx4d)


TBL_ROWS = OUT_PAD // NS
CB = C // 2


def _sc_gather_sp(table_bits, idx4d, b_pad):
    nchb = b_pad // (NW * CH)
    nbuf = 2
    nfull, tail = divmod(nchb, nbuf)

    def body(tbl, idx_hbm, out, idx_v, rows_v, tbl_sh, *sems):
        gsems, wsems = sems[:nbuf], sems[nbuf:]
        c = lax.axis_index("c")
        s = lax.axis_index("s")
        wid = s * NC + c
        rbase = wid * (nchb * CH)
        for q in range(TBL_ROWS // 112):
            r = s * TBL_ROWS + q * 112
            stage = rows_v.at[0].at[pl.ds(0, 112)]
            pltpu.sync_copy(tbl.at[pl.ds(r, 112)], stage)
            pltpu.sync_copy(stage, tbl_sh.at[pl.ds(r, 112)])
        pltpu.sync_copy(idx_hbm.at[wid], idx_v)
        plsc.subcore_barrier()
        for b in range(nbuf):
            pltpu.async_copy(tbl_sh.at[idx_v.at[b]], rows_v.at[b], gsems[b])

        def chunk(j, bb):
            pltpu.make_async_copy(tbl_sh.at[idx_v.at[j]], rows_v.at[bb],
                                  gsems[bb]).wait()
            dst = out.at[pl.ds(rbase + j * CH, CH)]
            pltpu.async_copy(rows_v.at[bb], dst, wsems[bb])
            pltpu.make_async_copy(rows_v.at[bb], dst, wsems[bb]).wait()
            jn = j + nbuf

            @pl.when(jn < nchb)
            def _():
                pltpu.async_copy(tbl_sh.at[idx_v.at[jn]], rows_v.at[bb],
                                 gsems[bb])

        def step(jb, carry):
            for b in range(nbuf):
                chunk(jb * nbuf + b, b)
            return carry

        lax.fori_loop(0, nfull, step, 0)
        for r in range(tail):
            chunk(nfull * nbuf + r, r)

    return pl.kernel(
        body,
        out_type=jax.ShapeDtypeStruct((b_pad, CB), jnp.int32),
        mesh=plsc.VectorSubcoreMesh(core_axis_name="c", subcore_axis_name="s"),
        compiler_params=pltpu.CompilerParams(use_tc_tiling_on_sc=False),
        scratch_types=[
            pltpu.VMEM((nchb, CH), jnp.int32),
            pltpu.VMEM((nbuf, CH, CB), jnp.int32),
            pltpu.VMEM_SHARED((OUT_PAD, CB), jnp.int32),
        ] + [pltpu.SemaphoreType.DMA] * (2 * nbuf),
    )(table_bits, idx4d)


SCCH = 64
IB = 8


def _sc_scatter(msgs, idx3d, zeros_tile, b_pad):
    nch = b_pad // (NS * SCCH)
    nbl_full, bl_tail = divmod(nch, IB)

    def body(msg_hbm, idx_hbm, zro, out, idx_v, li_v, msg_v, acc, *sems):
        lsems, msems, asems = sems[:2], sems[2:4], sems[4:6]
        c = lax.axis_index("c")
        s = lax.axis_index("s")
        row_base = c * HALF
        pltpu.sync_copy(zro, acc.at[pl.ds(s * PER_TILE, PER_TILE)])
        plsc.subcore_barrier()

        def idx_src(jb):
            return idx_hbm.at[s].at[pl.ds(jb * IB, IB)]

        def msg_src(j):
            return msg_hbm.at[pl.ds((s * nch + j) * SCCH, SCCH)]

        pltpu.async_copy(idx_src(0), idx_v.at[0], lsems[0])
        pltpu.async_copy(msg_src(0), msg_v.at[0], msems[0])
        pltpu.async_copy(msg_src(1), msg_v.at[1], msems[1])

        def block(jb, bl, nrows):
            pltpu.make_async_copy(idx_src(jb), idx_v.at[bl], lsems[bl]).wait()
            nb = jb + 1

            @pl.when(nb * IB < nch)
            def _():
                pltpu.async_copy(idx_src(nb), idx_v.at[1 - bl], lsems[1 - bl])

            for k in range(nrows):
                for t in range(SCCH // L):
                    v = idx_v[bl, k, pl.ds(t * L, L)]
                    li = v - row_base
                    ok = (li >= 0) & (li < HALF)
                    trash = HALF + (v & (NTRASH - 1))
                    li_v[bl, k, pl.ds(t * L, L)] = jnp.where(ok, li, trash)
            for k in range(nrows):
                j = jb * IB + k
                bb = k % 2
                pltpu.make_async_copy(msg_src(j), msg_v.at[bb],
                                      msems[bb]).wait()
                add = pltpu.async_copy(msg_v.at[bb],
                                       acc.at[li_v.at[bl].at[k]],
                                       asems[bb], add=True)
                add.wait()
                jn = j + 2

                @pl.when(jn < nch)
                def _():
                    pltpu.async_copy(msg_src(jn), msg_v.at[bb], msems[bb])

        def step2(jp, carry):
            block(jp * 2, 0, IB)
            block(jp * 2 + 1, 1, IB)
            return carry

        npair = nbl_full // 2
        lax.fori_loop(0, npair, step2, 0)
        if nbl_full % 2:
            block(nbl_full - 1, (nbl_full - 1) % 2, IB)
        if bl_tail:
            block(nbl_full, nbl_full % 2, bl_tail)
        plsc.subcore_barrier()
        nwb, wb_tail = divmod(PER_TILE, SCCH)
        for q in range(nwb):
            r = s * PER_TILE + q * SCCH
            pltpu.sync_copy(acc.at[pl.ds(r, SCCH)], msg_v.at[0])
            pltpu.sync_copy(msg_v.at[0], out.at[pl.ds(row_base + r, SCCH)])
        if wb_tail:
            r = s * PER_TILE + nwb * SCCH
            pltpu.sync_copy(acc.at[pl.ds(r, wb_tail)],
                            msg_v.at[0].at[pl.ds(0, wb_tail)])
            pltpu.sync_copy(msg_v.at[0].at[pl.ds(0, wb_tail)],
                            out.at[pl.ds(row_base + r, wb_tail)])

    return pl.kernel(
        body,
        out_type=jax.ShapeDtypeStruct((OUT_PAD, C), jnp.float32),
        mesh=plsc.VectorSubcoreMesh(core_axis_name="c", subcore_axis_name="s"),
        scratch_types=[
            pltpu.VMEM((2, IB, SCCH), jnp.int32),
            pltpu.VMEM((2, IB, SCCH), jnp.int32),
            pltpu.VMEM((2, SCCH, C), jnp.float32),
            pltpu.VMEM_SHARED((ACC_ROWS, C), jnp.float32),
        ] + [pltpu.SemaphoreType.DMA] * 6,
    )(msgs, idx3d, zeros_tile)


def _to_bits(t):
    b = t.astype(jnp.bfloat16).reshape(t.shape[0], C // 2, 2)
    return lax.bitcast_convert_type(b, jnp.int32)


def _from_bits(g, b_pad):
    return lax.bitcast_convert_type(g, jnp.bfloat16).reshape(b_pad, C)


def _tc_stats(raw, gb, emit_bits=False):
    nblk = OUT_PAD // BLK

    def body(raw_ref, gb_ref, *refs):
        if emit_bits:
            out_ref, bits_ref, acc_ref = refs
        else:
            out_ref, acc_ref = refs
        b = pl.program_id(0)

        @pl.when(b == 0)
        def _():
            acc_ref[...] = jnp.zeros_like(acc_ref)

        x = raw_ref[...]
        if emit_bits:
            h = C // 2
            rn_lo = x[:, :h].astype(jnp.bfloat16).astype(jnp.float32)
            rn_hi = x[:, h:].astype(jnp.bfloat16).astype(jnp.float32)
            lo = lax.shift_right_logical(
                lax.bitcast_convert_type(rn_lo, jnp.int32), 16)
            hi = lax.bitcast_convert_type(rn_hi, jnp.int32) & jnp.int32(-65536)
            bits_ref[...] = lo | hi
        rows = b * BLK + lax.broadcasted_iota(jnp.int32, (BLK, C), 0)
        xm = jnp.where(rows < NOUT, x, 0.0)
        acc_ref[0:1, :] += jnp.sum(xm, axis=0, keepdims=True)
        acc_ref[1:2, :] += jnp.sum(xm * xm, axis=0, keepdims=True)

        @pl.when(b == nblk - 1)
        def _():
            mean = acc_ref[0:1, :] / NOUT
            var = acc_ref[1:2, :] / NOUT - mean * mean
            scale = gb_ref[0:1, :] * lax.rsqrt(var + 1e-5)
            shift = gb_ref[1:2, :] - mean * scale
            out_ref[...] = jnp.concatenate(
                [scale, shift, jnp.zeros((6, C), jnp.float32)], axis=0)

    out_specs = [pl.BlockSpec((8, C), lambda b: (0, 0))]
    out_shape = [jax.ShapeDtypeStruct((8, C), jnp.float32)]
    if emit_bits:
        out_specs.append(pl.BlockSpec((BLK, C // 2), lambda b: (b, 0)))
        out_shape.append(jax.ShapeDtypeStruct((OUT_PAD, C // 2), jnp.int32))
    res = pl.pallas_call(
        body,
        grid=(nblk,),
        in_specs=[pl.BlockSpec((BLK, C), lambda b: (b, 0)),
                  pl.BlockSpec((8, C), lambda b: (0, 0))],
        out_specs=out_specs,
        out_shape=out_shape,
        scratch_shapes=[pltpu.VMEM((8, C), jnp.float32)],
    )(raw, gb)
    return res if emit_bits else res[0]


def _tc_matmul(G, W, st, bpk, apply_act):
    B = G.shape[0]
    K = W.shape[0]
    nblk = B // BLK

    def body(g_ref, w_ref, st_ref, o_ref):
        g = g_ref[...].astype(jnp.float32)
        if apply_act:
            g = jnp.maximum(g * st_ref[0:1, :] + st_ref[1:2, :], 0.0)
        o_ref[...] = jnp.dot(g, w_ref[0], preferred_element_type=jnp.float32)

    return pl.pallas_call(
        body,
        grid=(nblk,),
        in_specs=[pl.BlockSpec((BLK, C), lambda b: (b, 0)),
                  pl.BlockSpec((1, C, C),
                               lambda b: (jnp.minimum(b // bpk, K - 1), 0, 0)),
                  pl.BlockSpec((8, C), lambda b: (0, 0))],
        out_specs=pl.BlockSpec((BLK, C), lambda b: (b, 0)),
        out_shape=jax.ShapeDtypeStruct((B, C), jnp.float32),
    )(G, W, st)


def _tc_matmul_packed(gbits, wp, stp, bpk):
    B = gbits.shape[0]
    K = wp.shape[0]
    nblk = B // BLK

    def body(g_ref, w_ref, st_ref, o_ref):
        x = g_ref[...]
        lo = lax.bitcast_convert_type(lax.shift_left(x, 16), jnp.float32)
        hi = lax.bitcast_convert_type(x & jnp.int32(-65536), jnp.float32)
        ge = jnp.maximum(lo * st_ref[0:1, 0:64] + st_ref[1:2, 0:64], 0.0)
        go = jnp.maximum(hi * st_ref[2:3, 0:64] + st_ref[3:4, 0:64], 0.0)
        o_ref[...] = (
            jnp.dot(ge, w_ref[0, 0], preferred_element_type=jnp.float32)
            + jnp.dot(go, w_ref[0, 1], preferred_element_type=jnp.float32))

    return pl.pallas_call(
        body,
        grid=(nblk,),
        in_specs=[pl.BlockSpec((BLK, C // 2), lambda b: (b, 0)),
                  pl.BlockSpec((1, 2, C // 2, C),
                               lambda b: (jnp.minimum(b // bpk, K - 1), 0, 0, 0)),
                  pl.BlockSpec((8, C), lambda b: (0, 0))],
        out_specs=pl.BlockSpec((BLK, C), lambda b: (b, 0)),
        out_shape=jax.ShapeDtypeStruct((B, C), jnp.float32),
    )(gbits, wp, stp)


def _split_w(W):
    h = C // 2
    return jnp.stack([W[:, :h, :], W[:, h:, :]], axis=1)


def _split_st(st):
    h = C // 2
    top = jnp.stack([st[0, :h], st[1, :h], st[0, h:], st[1, h:]])
    return jnp.pad(top, ((0, 4), (0, h)))


def _tc_final(raw2, P, st2, stp):
    nblk = OUT_PAD // BLK

    def body(a_ref, p_ref, s2_ref, sp_ref, o_ref):
        a = a_ref[...] * s2_ref[0:1, :] + s2_ref[1:2, :]
        q = p_ref[...] * sp_ref[0:1, :] + sp_ref[1:2, :]
        o_ref[...] = jnp.maximum(a + q, 0.0)

    return pl.pallas_call(
        body,
        grid=(nblk,),
        in_specs=[pl.BlockSpec((BLK, C), lambda b: (b, 0)),
                  pl.BlockSpec((BLK, C), lambda b: (b, 0)),
                  pl.BlockSpec((8, C), lambda b: (0, 0)),
                  pl.BlockSpec((8, C), lambda b: (0, 0))],
        out_specs=pl.BlockSpec((BLK, C), lambda b: (b, 0)),
        out_shape=jax.ShapeDtypeStruct((OUT_PAD, C), jnp.float32),
    )(raw2, P, st2, stp)


def _pad_idx(idx, e_pad, b_pad, fill, nw, width=CH, row_align=1):
    k, e = idx.shape
    p = jnp.pad(idx.astype(jnp.int32), ((0, 0), (0, e_pad - e)),
                constant_values=fill)
    flat = p.reshape(-1)
    flat = jnp.pad(flat, (0, b_pad - flat.shape[0]), constant_values=fill)
    a = flat.reshape(nw, b_pad // (nw * width), width)
    nch = a.shape[1]
    nch_pad = -(-nch // row_align) * row_align
    if nch_pad != nch:
        a = jnp.pad(a, ((0, 0), (0, nch_pad - nch), (0, 0)),
                    constant_values=fill)
    return a


def _pad_idx_g(idx, e_pad, b_pad, fill, br):
    a = _pad_idx(idx, e_pad, b_pad, fill, NW)
    return a.reshape(NW, a.shape[1] // br, br * CH)


def _gb(g, b):
    return jnp.concatenate([g[None], b[None], jnp.zeros((6, C), jnp.float32)], 0)


def kernel(x, down_in_idx, down_out_idx, sub_in_idx, sub_out_idx,
           W_down, W1, W2, W_proj,
           g_down, b_down, g1, b1, g2, b2, g_proj, b_proj):
    din = _pad_idx_g(down_in_idx, ED_PAD, BD, 0, br=1)
    dout = _pad_idx(down_out_idx, ED_PAD, BD, NOUT, NS, width=SCCH,
                    row_align=IB)
    sin = _pad_idx_g(sub_in_idx, ES_PAD, BS, 0, br=1)
    sout = _pad_idx(sub_out_idx, ES_PAD, BS, NOUT, NS, width=SCCH,
                    row_align=IB)
    zeros_tile = jnp.zeros((PER_TILE, C), jnp.float32)
    st0 = jnp.zeros((8, C), jnp.float32)

    Gd = _sc_gather(x, din, BD, br=1, nbuf=4)
    Md = _tc_matmul(Gd, W_down, st0, ED_PAD // BLK, apply_act=False)
    raw_h = _sc_scatter(Md, dout, zeros_tile, BD)
    st_h, bits_h = _tc_stats(raw_h, _gb(g_down, b_down), emit_bits=True)

    G1 = _sc_gather_sp(bits_h, sin, BS)
    M1 = _tc_matmul_packed(G1, _split_w(W1), _split_st(st_h), ES_PAD // BLK)
    raw1 = _sc_scatter(M1, sout, zeros_tile, BS)
    st_1, bits1 = _tc_stats(raw1, _gb(g1, b1), emit_bits=True)

    G2 = _sc_gather_sp(bits1, sin, BS)
    M2 = _tc_matmul_packed(G2, _split_w(W2), _split_st(st_1), ES_PAD // BLK)
    raw2 = _sc_scatter(M2, sout, zeros_tile, BS)
    st_2 = _tc_stats(raw2, _gb(g2, b2))

    P = _tc_matmul(raw_h, W_proj[None], st_h, OUT_PAD // BLK, apply_act=True)
    st_p = _tc_stats(P, _gb(g_proj, b_proj))

    outp = _tc_final(raw2, P, st_2, st_p)
    return outp[:NOUT]


# speedup vs baseline: 2.3420x; 1.0299x over previous
import jax
import jax.numpy as jnp
from jax import lax
from jax.experimental import pallas as pl
from jax.experimental.pallas import tpu as pltpu
from jax.experimental.pallas import tpu_sc as plsc

C = 128
NOUT = 25000
HALF = 12544
OUT_PAD = 2 * HALF
NTRASH = 64
ACC_ROWS = HALF + NTRASH
PER_TILE = HALF // 16
NC, NS, L = 2, 16, 16
NW = NC * NS
CH = 128
BLK = 512

KD, ED, ED_PAD = 8, 25000, 25088
BD = KD * ED_PAD
KS, ES, ES_PAD = 27, 15000, 15360
BS = 417792


def _sc_gather(table, idx4d, b_pad, br, nbuf, cols=C, dtype=jnp.float32):
    brch = br * CH
    nchb = b_pad // (NW * brch)
    nfull, tail = divmod(nchb, nbuf)

    def body(tbl, idx_hbm, out, idx_v, rows_v, *sems):
        gsems, wsems = sems[:nbuf], sems[nbuf:]
        c = lax.axis_index("c")
        s = lax.axis_index("s")
        wid = s * NC + c
        rbase = wid * (nchb * brch)
        pltpu.sync_copy(idx_hbm.at[wid], idx_v)
        for b in range(min(nbuf, nchb)):
            pltpu.async_copy(tbl.at[idx_v.at[b]], rows_v.at[b], gsems[b])

        def chunk(j, bb):
            pltpu.make_async_copy(tbl.at[idx_v.at[j]], rows_v.at[bb],
                                  gsems[bb]).wait()
            dst = out.at[pl.ds(rbase + j * brch, brch)]
            pltpu.async_copy(rows_v.at[bb], dst, wsems[bb])
            pltpu.make_async_copy(rows_v.at[bb], dst, wsems[bb]).wait()
            jn = j + nbuf

            @pl.when(jn < nchb)
            def _():
                pltpu.async_copy(tbl.at[idx_v.at[jn]], rows_v.at[bb],
                                 gsems[bb])

        def step(jb, carry):
            for b in range(nbuf):
                chunk(jb * nbuf + b, b)
            return carry

        lax.fori_loop(0, nfull, step, 0)
        for r in range(tail):
            chunk(nfull * nbuf + r, r)

    return pl.kernel(
        body,
        out_type=jax.ShapeDtypeStruct((b_pad, cols), dtype),
        mesh=plsc.VectorSubcoreMesh(core_axis_name="c", subcore_axis_name="s"),
        scratch_types=[
            pltpu.VMEM((nchb, brch), jnp.int32),
            pltpu.VMEM((nbuf, brch, cols), dtype),
        ] + [pltpu.SemaphoreType.DMA] * (2 * nbuf),
    )(table, idx4d)


TBL_ROWS = OUT_PAD // NS
CB = C // 2


def _sc_gather_sp(table_bits, idx4d, b_pad):
    nchb = b_pad // (NW * CH)
    nbuf = 2
    nfull, tail = divmod(nchb, nbuf)

    def body(tbl, idx_hbm, out, idx_v, rows_v, tbl_sh, *sems):
        gsems, wsems = sems[:nbuf], sems[nbuf:]
        c = lax.axis_index("c")
        s = lax.axis_index("s")
        wid = s * NC + c
        rbase = wid * (nchb * CH)
        for q in range(TBL_ROWS // 112):
            r = s * TBL_ROWS + q * 112
            stage = rows_v.at[0].at[pl.ds(0, 112)]
            pltpu.sync_copy(tbl.at[pl.ds(r, 112)], stage)
            pltpu.sync_copy(stage, tbl_sh.at[pl.ds(r, 112)])
        pltpu.sync_copy(idx_hbm.at[wid], idx_v)
        plsc.subcore_barrier()
        for b in range(nbuf):
            pltpu.async_copy(tbl_sh.at[idx_v.at[b]], rows_v.at[b], gsems[b])

        def chunk(j, bb):
            pltpu.make_async_copy(tbl_sh.at[idx_v.at[j]], rows_v.at[bb],
                                  gsems[bb]).wait()
            dst = out.at[pl.ds(rbase + j * CH, CH)]
            pltpu.async_copy(rows_v.at[bb], dst, wsems[bb])
            pltpu.make_async_copy(rows_v.at[bb], dst, wsems[bb]).wait()
            jn = j + nbuf

            @pl.when(jn < nchb)
            def _():
                pltpu.async_copy(tbl_sh.at[idx_v.at[jn]], rows_v.at[bb],
                                 gsems[bb])

        def step(jb, carry):
            for b in range(nbuf):
                chunk(jb * nbuf + b, b)
            return carry

        lax.fori_loop(0, nfull, step, 0)
        for r in range(tail):
            chunk(nfull * nbuf + r, r)

    return pl.kernel(
        body,
        out_type=jax.ShapeDtypeStruct((b_pad, CB), jnp.int32),
        mesh=plsc.VectorSubcoreMesh(core_axis_name="c", subcore_axis_name="s"),
        compiler_params=pltpu.CompilerParams(use_tc_tiling_on_sc=False),
        scratch_types=[
            pltpu.VMEM((nchb, CH), jnp.int32),
            pltpu.VMEM((nbuf, CH, CB), jnp.int32),
            pltpu.VMEM_SHARED((OUT_PAD, CB), jnp.int32),
        ] + [pltpu.SemaphoreType.DMA] * (2 * nbuf),
    )(table_bits, idx4d)


SCCH = 64
IB = 8


def _sc_scatter(msgs, idx3d, zeros_tile, b_pad, scch=SCCH):
    nch = b_pad // (NS * scch)
    nbl_full, bl_tail = divmod(nch, IB)

    def body(msg_hbm, idx_hbm, zro, out, idx_v, li_v, msg_v, acc, *sems):
        lsems, msems, asems = sems[:2], sems[2:4], sems[4:6]
        c = lax.axis_index("c")
        s = lax.axis_index("s")
        row_base = c * HALF
        pltpu.sync_copy(zro, acc.at[pl.ds(s * PER_TILE, PER_TILE)])
        plsc.subcore_barrier()

        def idx_src(jb):
            return idx_hbm.at[s].at[pl.ds(jb * IB, IB)]

        def msg_src(j):
            return msg_hbm.at[pl.ds((s * nch + j) * scch, scch)]

        pltpu.async_copy(idx_src(0), idx_v.at[0], lsems[0])
        pltpu.async_copy(msg_src(0), msg_v.at[0], msems[0])
        pltpu.async_copy(msg_src(1), msg_v.at[1], msems[1])

        def block(jb, bl, nrows):
            pltpu.make_async_copy(idx_src(jb), idx_v.at[bl], lsems[bl]).wait()
            nb = jb + 1

            @pl.when(nb * IB < nch)
            def _():
                pltpu.async_copy(idx_src(nb), idx_v.at[1 - bl], lsems[1 - bl])

            for k in range(nrows):
                for t in range(scch // L):
                    v = idx_v[bl, k, pl.ds(t * L, L)]
                    li = v - row_base
                    ok = (li >= 0) & (li < HALF)
                    trash = HALF + (v & (NTRASH - 1))
                    li_v[bl, k, pl.ds(t * L, L)] = jnp.where(ok, li, trash)
            for k in range(nrows):
                j = jb * IB + k
                bb = k % 2
                pltpu.make_async_copy(msg_src(j), msg_v.at[bb],
                                      msems[bb]).wait()
                add = pltpu.async_copy(msg_v.at[bb],
                                       acc.at[li_v.at[bl].at[k]],
                                       asems[bb], add=True)
                add.wait()
                jn = j + 2

                @pl.when(jn < nch)
                def _():
                    pltpu.async_copy(msg_src(jn), msg_v.at[bb], msems[bb])

        def step2(jp, carry):
            block(jp * 2, 0, IB)
            block(jp * 2 + 1, 1, IB)
            return carry

        npair = nbl_full // 2
        lax.fori_loop(0, npair, step2, 0)
        if nbl_full % 2:
            block(nbl_full - 1, (nbl_full - 1) % 2, IB)
        if bl_tail:
            block(nbl_full, nbl_full % 2, bl_tail)
        plsc.subcore_barrier()
        nwb, wb_tail = divmod(PER_TILE, scch)
        for q in range(nwb):
            r = s * PER_TILE + q * scch
            pltpu.sync_copy(acc.at[pl.ds(r, scch)], msg_v.at[0])
            pltpu.sync_copy(msg_v.at[0], out.at[pl.ds(row_base + r, scch)])
        if wb_tail:
            r = s * PER_TILE + nwb * SCCH
            pltpu.sync_copy(acc.at[pl.ds(r, wb_tail)],
                            msg_v.at[0].at[pl.ds(0, wb_tail)])
            pltpu.sync_copy(msg_v.at[0].at[pl.ds(0, wb_tail)],
                            out.at[pl.ds(row_base + r, wb_tail)])

    return pl.kernel(
        body,
        out_type=jax.ShapeDtypeStruct((OUT_PAD, C), jnp.float32),
        mesh=plsc.VectorSubcoreMesh(core_axis_name="c", subcore_axis_name="s"),
        scratch_types=[
            pltpu.VMEM((2, IB, scch), jnp.int32),
            pltpu.VMEM((2, IB, scch), jnp.int32),
            pltpu.VMEM((2, scch, C), jnp.float32),
            pltpu.VMEM_SHARED((ACC_ROWS, C), jnp.float32),
        ] + [pltpu.SemaphoreType.DMA] * 6,
    )(msgs, idx3d, zeros_tile)


def _to_bits(t):
    b = t.astype(jnp.bfloat16).reshape(t.shape[0], C // 2, 2)
    return lax.bitcast_convert_type(b, jnp.int32)


def _from_bits(g, b_pad):
    return lax.bitcast_convert_type(g, jnp.bfloat16).reshape(b_pad, C)


def _tc_stats(raw, gb, emit_bits=False):
    nblk = OUT_PAD // BLK

    def body(raw_ref, gb_ref, *refs):
        if emit_bits:
            out_ref, bits_ref, acc_ref = refs
        else:
            out_ref, acc_ref = refs
        b = pl.program_id(0)

        @pl.when(b == 0)
        def _():
            acc_ref[...] = jnp.zeros_like(acc_ref)

        x = raw_ref[...]
        if emit_bits:
            h = C // 2
            rn_lo = x[:, :h].astype(jnp.bfloat16).astype(jnp.float32)
            rn_hi = x[:, h:].astype(jnp.bfloat16).astype(jnp.float32)
            lo = lax.shift_right_logical(
                lax.bitcast_convert_type(rn_lo, jnp.int32), 16)
            hi = lax.bitcast_convert_type(rn_hi, jnp.int32) & jnp.int32(-65536)
            bits_ref[...] = lo | hi
        rows = b * BLK + lax.broadcasted_iota(jnp.int32, (BLK, C), 0)
        xm = jnp.where(rows < NOUT, x, 0.0)
        acc_ref[0:1, :] += jnp.sum(xm, axis=0, keepdims=True)
        acc_ref[1:2, :] += jnp.sum(xm * xm, axis=0, keepdims=True)

        @pl.when(b == nblk - 1)
        def _():
            mean = acc_ref[0:1, :] / NOUT
            var = acc_ref[1:2, :] / NOUT - mean * mean
            scale = gb_ref[0:1, :] * lax.rsqrt(var + 1e-5)
            shift = gb_ref[1:2, :] - mean * scale
            out_ref[...] = jnp.concatenate(
                [scale, shift, jnp.zeros((6, C), jnp.float32)], axis=0)

    out_specs = [pl.BlockSpec((8, C), lambda b: (0, 0))]
    out_shape = [jax.ShapeDtypeStruct((8, C), jnp.float32)]
    if emit_bits:
        out_specs.append(pl.BlockSpec((BLK, C // 2), lambda b: (b, 0)))
        out_shape.append(jax.ShapeDtypeStruct((OUT_PAD, C // 2), jnp.int32))
    res = pl.pallas_call(
        body,
        grid=(nblk,),
        in_specs=[pl.BlockSpec((BLK, C), lambda b: (b, 0)),
                  pl.BlockSpec((8, C), lambda b: (0, 0))],
        out_specs=out_specs,
        out_shape=out_shape,
        scratch_shapes=[pltpu.VMEM((8, C), jnp.float32)],
    )(raw, gb)
    return res if emit_bits else res[0]


def _tc_matmul(G, W, st, bpk, apply_act):
    B = G.shape[0]
    K = W.shape[0]
    nblk = B // BLK

    def body(g_ref, w_ref, st_ref, o_ref):
        g = g_ref[...].astype(jnp.float32)
        if apply_act:
            g = jnp.maximum(g * st_ref[0:1, :] + st_ref[1:2, :], 0.0)
        o_ref[...] = jnp.dot(g, w_ref[0], preferred_element_type=jnp.float32)

    return pl.pallas_call(
        body,
        grid=(nblk,),
        in_specs=[pl.BlockSpec((BLK, C), lambda b: (b, 0)),
                  pl.BlockSpec((1, C, C),
                               lambda b: (jnp.minimum(b // bpk, K - 1), 0, 0)),
                  pl.BlockSpec((8, C), lambda b: (0, 0))],
        out_specs=pl.BlockSpec((BLK, C), lambda b: (b, 0)),
        out_shape=jax.ShapeDtypeStruct((B, C), jnp.float32),
    )(G, W, st)


def _tc_matmul_packed(gbits, wp, stp, bpk):
    B = gbits.shape[0]
    K = wp.shape[0]
    nblk = B // BLK

    def body(g_ref, w_ref, st_ref, o_ref):
        x = g_ref[...]
        lo = lax.bitcast_convert_type(lax.shift_left(x, 16), jnp.float32)
        hi = lax.bitcast_convert_type(x & jnp.int32(-65536), jnp.float32)
        ge = jnp.maximum(lo * st_ref[0:1, 0:64] + st_ref[1:2, 0:64], 0.0)
        go = jnp.maximum(hi * st_ref[2:3, 0:64] + st_ref[3:4, 0:64], 0.0)
        o_ref[...] = (
            jnp.dot(ge, w_ref[0, 0], preferred_element_type=jnp.float32)
            + jnp.dot(go, w_ref[0, 1], preferred_element_type=jnp.float32))

    return pl.pallas_call(
        body,
        grid=(nblk,),
        in_specs=[pl.BlockSpec((BLK, C // 2), lambda b: (b, 0)),
                  pl.BlockSpec((1, 2, C // 2, C),
                               lambda b: (jnp.minimum(b // bpk, K - 1), 0, 0, 0)),
                  pl.BlockSpec((8, C), lambda b: (0, 0))],
        out_specs=pl.BlockSpec((BLK, C), lambda b: (b, 0)),
        out_shape=jax.ShapeDtypeStruct((B, C), jnp.float32),
    )(gbits, wp, stp)


def _split_w(W):
    h = C // 2
    return jnp.stack([W[:, :h, :], W[:, h:, :]], axis=1)


def _split_st(st):
    h = C // 2
    top = jnp.stack([st[0, :h], st[1, :h], st[0, h:], st[1, h:]])
    return jnp.pad(top, ((0, 4), (0, h)))


def _tc_final(raw2, P, st2, stp):
    nblk = OUT_PAD // BLK

    def body(a_ref, p_ref, s2_ref, sp_ref, o_ref):
        a = a_ref[...] * s2_ref[0:1, :] + s2_ref[1:2, :]
        q = p_ref[...] * sp_ref[0:1, :] + sp_ref[1:2, :]
        o_ref[...] = jnp.maximum(a + q, 0.0)

    return pl.pallas_call(
        body,
        grid=(nblk,),
        in_specs=[pl.BlockSpec((BLK, C), lambda b: (b, 0)),
                  pl.BlockSpec((BLK, C), lambda b: (b, 0)),
                  pl.BlockSpec((8, C), lambda b: (0, 0)),
                  pl.BlockSpec((8, C), lambda b: (0, 0))],
        out_specs=pl.BlockSpec((BLK, C), lambda b: (b, 0)),
        out_shape=jax.ShapeDtypeStruct((OUT_PAD, C), jnp.float32),
    )(raw2, P, st2, stp)


def _pad_idx(idx, e_pad, b_pad, fill, nw, width=CH, row_align=1):
    k, e = idx.shape
    p = jnp.pad(idx.astype(jnp.int32), ((0, 0), (0, e_pad - e)),
                constant_values=fill)
    flat = p.reshape(-1)
    flat = jnp.pad(flat, (0, b_pad - flat.shape[0]), constant_values=fill)
    a = flat.reshape(nw, b_pad // (nw * width), width)
    nch = a.shape[1]
    nch_pad = -(-nch // row_align) * row_align
    if nch_pad != nch:
        a = jnp.pad(a, ((0, 0), (0, nch_pad - nch), (0, 0)),
                    constant_values=fill)
    return a


def _pad_idx_g(idx, e_pad, b_pad, fill, br):
    a = _pad_idx(idx, e_pad, b_pad, fill, NW)
    return a.reshape(NW, a.shape[1] // br, br * CH)


def _gb(g, b):
    return jnp.concatenate([g[None], b[None], jnp.zeros((6, C), jnp.float32)], 0)


def kernel(x, down_in_idx, down_out_idx, sub_in_idx, sub_out_idx,
           W_down, W1, W2, W_proj,
           g_down, b_down, g1, b1, g2, b2, g_proj, b_proj):
    din = _pad_idx_g(down_in_idx, ED_PAD, BD, 0, br=1)
    dout = _pad_idx(down_out_idx, ED_PAD, BD, NOUT, NS, width=SCCH,
                    row_align=IB)
    sin = _pad_idx_g(sub_in_idx, ES_PAD, BS, 0, br=1)
    sout = _pad_idx(sub_out_idx, ES_PAD, BS, NOUT, NS, width=96,
                    row_align=IB)
    zeros_tile = jnp.zeros((PER_TILE, C), jnp.float32)
    st0 = jnp.zeros((8, C), jnp.float32)

    Gd = _sc_gather(x, din, BD, br=1, nbuf=4)
    Md = _tc_matmul(Gd, W_down, st0, ED_PAD // BLK, apply_act=False)
    raw_h = _sc_scatter(Md, dout, zeros_tile, BD)
    st_h, bits_h = _tc_stats(raw_h, _gb(g_down, b_down), emit_bits=True)

    G1 = _sc_gather_sp(bits_h, sin, BS)
    M1 = _tc_matmul_packed(G1, _split_w(W1), _split_st(st_h), ES_PAD // BLK)
    raw1 = _sc_scatter(M1, sout, zeros_tile, BS, scch=96)
    st_1, bits1 = _tc_stats(raw1, _gb(g1, b1), emit_bits=True)

    G2 = _sc_gather_sp(bits1, sin, BS)
    M2 = _tc_matmul_packed(G2, _split_w(W2), _split_st(st_1), ES_PAD // BLK)
    raw2 = _sc_scatter(M2, sout, zeros_tile, BS, scch=96)
    st_2 = _tc_stats(raw2, _gb(g2, b2))

    P = _tc_matmul(raw_h, W_proj[None], st_h, OUT_PAD // BLK, apply_act=True)
    st_p = _tc_stats(P, _gb(g_proj, b_proj))

    outp = _tc_final(raw2, P, st_2, st_p)
    return outp[:NOUT]
